# baseline (speedup 1.0000x reference)
import functools

import jax
import jax.numpy as jnp
from jax import lax
from jax.experimental import pallas as pl
from jax.experimental.pallas import tpu as pltpu
from jax.experimental.pallas import tpu_sc as plsc

N = 10000
E = 320000
T = 8
F_IN = 128
H = 64

NP = 10112
EP = 327680
NCHUNK = 8
CW = 64
CPS = NCHUNK // 2
E_ROWS = EP // 128
NSUB = 16
RPT = NP // NSUB
ER_TILE = E_ROWS // NSUB
KROWS = 1
GROUPS = ER_TILE // KROWS
GB = KROWS * 128
SEGS = 2
SEG_GROUPS = GROUPS // SEGS
SEG_ROWS = SEG_GROUPS * KROWS

BN = 1000
NBLK = N // BN

_f32 = jnp.float32


def _make_sc_agg(with_cnt):
  mesh = plsc.VectorSubcoreMesh(core_axis_name="c", subcore_axis_name="s")
  out_type = [jax.ShapeDtypeStruct((NCHUNK * NP, CW), _f32)]
  scratch = [
      pltpu.VMEM((SEG_ROWS, 128), jnp.int32),
      pltpu.VMEM((SEG_ROWS, 128), jnp.int32),
      pltpu.VMEM((GB, CW), _f32),
      pltpu.VMEM((GB, CW), _f32),
      pltpu.VMEM_SHARED((NP, CW), _f32),
      pltpu.SemaphoreType.DMA,
      pltpu.SemaphoreType.DMA,
      pltpu.SemaphoreType.DMA,
      pltpu.SemaphoreType.DMA,
  ]
  if with_cnt:
    out_type.append(jax.ShapeDtypeStruct((2 * NP, 16), _f32))
    scratch += [
        pltpu.VMEM((128, 16), _f32),
        pltpu.VMEM_SHARED((NP, 16), _f32),
    ]

  def body(y_h, s4_h, d2_h, z2_h, zc_h, o1_h, *rest):
    if with_cnt:
      (out_h, cnt_h, src_sl, dst_sl, rows0, rows1, acc_sp,
       gs0, gs1, ss0, ss1, o1_v, cnt_sp) = rest
    else:
      out_h, src_sl, dst_sl, rows0, rows1, acc_sp, gs0, gs1, ss0, ss1 = rest
    rows = (rows0, rows1)
    gsem = (gs0, gs1)
    cid = lax.axis_index("c")
    sid = lax.axis_index("s")
    row0 = sid * RPT
    if with_cnt:
      pltpu.sync_copy(o1_h, o1_v)
      pltpu.sync_copy(zc_h.at[pl.ds(row0, RPT)], cnt_sp.at[pl.ds(row0, RPT)])

    def fire_gathers(g, b):
      pltpu.async_copy(y_h.at[src_sl.at[g, :]], rows[b], gsem[b])

    for q in range(CPS):
      chunk = cid * CPS + q
      pltpu.sync_copy(z2_h.at[pl.ds(row0, RPT)], acc_sp.at[pl.ds(row0, RPT)])
      plsc.subcore_barrier()

      for seg in range(SEGS):
        pltpu.sync_copy(
            s4_h.at[pl.ds(chunk * E_ROWS + sid * ER_TILE + seg * SEG_ROWS,
                          SEG_ROWS)], src_sl)
        pltpu.sync_copy(
            d2_h.at[pl.ds(sid * ER_TILE + seg * SEG_ROWS, SEG_ROWS)], dst_sl)

        fire_gathers(0, 0)
        fire_gathers(1, 1)

        def pair_body(gg, carry):
          for b in range(2):
            g = gg * 2 + b
            pltpu.make_async_copy(y_h.at[pl.ds(0, GB)], rows[b],
                                  gsem[b]).wait()
            pltpu.sync_copy(rows[b], acc_sp.at[dst_sl.at[g, :]], add=True)
            if with_cnt and q == 0:
              pltpu.sync_copy(o1_v, cnt_sp.at[dst_sl.at[g, :]], add=True)

            @pl.when(g + 2 < SEG_GROUPS)
            def _():
              fire_gathers(g + 2, b)
          return carry

        lax.fori_loop(0, SEG_GROUPS // 2, pair_body, 0)
      plsc.subcore_barrier()
      pltpu.sync_copy(acc_sp.at[pl.ds(row0, RPT)],
                      out_h.at[pl.ds(chunk * NP + row0, RPT)])
      if with_cnt and q == 0:
        pltpu.sync_copy(cnt_sp.at[pl.ds(row0, RPT)],
                        cnt_h.at[pl.ds(cid * NP + row0, RPT)])

  return functools.partial(
      pl.kernel, mesh=mesh, out_type=out_type, scratch_types=scratch,
      compiler_params=pltpu.CompilerParams(use_tc_tiling_on_sc=False))(body)


@functools.lru_cache(maxsize=None)
def _sc_agg_fn(with_cnt):
  return _make_sc_agg(with_cnt)


def _sc_agg_cnt(*args):
  return _sc_agg_fn(True)(*args)


def _sc_agg(*args):
  out = _sc_agg_fn(False)(*args)
  return out[0] if isinstance(out, (list, tuple)) else out


def _stage1_body(x_ref, wl_ref, wr_ref, y_ref, r_ref):
  for t in range(T):
    xt = x_ref[t]
    y_ref[t] = jnp.dot(xt, wl_ref[...], preferred_element_type=_f32)
    r_ref[:, t * H:(t + 1) * H] = jnp.dot(
        xt, wr_ref[...], preferred_element_type=_f32)


def _stage1(x_seq, Wl1, Wr1):
  return pl.pallas_call(
      _stage1_body,
      grid=(NBLK,),
      in_specs=[
          pl.BlockSpec((T, BN, F_IN), lambda i: (0, i, 0)),
          pl.BlockSpec((F_IN, H), lambda i: (0, 0)),
          pl.BlockSpec((F_IN, H), lambda i: (0, 0)),
      ],
      out_specs=[
          pl.BlockSpec((NCHUNK, BN, CW), lambda i: (0, i, 0)),
          pl.BlockSpec((BN, T * H), lambda i: (i, 0)),
      ],
      out_shape=[
          jax.ShapeDtypeStruct((NCHUNK, NP, CW), _f32),
          jax.ShapeDtypeStruct((N, T * H), _f32),
      ],
  )(x_seq, Wl1, Wr1)


def _inv_cnt(cntp):
  cnt16 = jnp.sum(cntp, axis=1)
  return (16.0 / jnp.maximum(cnt16, 16.0))[:, None]


def _ln_relu(pre, g_ref, b_ref):
  mu = jnp.mean(pre, axis=1, keepdims=True)
  var = jnp.mean((pre - mu) * (pre - mu), axis=1, keepdims=True)
  hn = (pre - mu) * lax.rsqrt(var + 1e-5) * g_ref[...] + b_ref[...]
  return jnp.maximum(hn, 0.0)


def _stage3_body(a_ref, r_ref, cntp_ref, wl2_ref, wr2_ref, bl1_ref, g1_ref,
                 be1_ref, h_ref, y2_ref, r2_ref):
  inv = _inv_cnt(cntp_ref[...])
  for t in range(T):
    pre = (a_ref[t] * inv + bl1_ref[...]
           + r_ref[:, t * H:(t + 1) * H])
    h = _ln_relu(pre, g1_ref, be1_ref)
    h_ref[:, t * H:(t + 1) * H] = h
    y2_ref[t] = jnp.dot(h, wl2_ref[...], preferred_element_type=_f32)
    r2_ref[:, t * H:(t + 1) * H] = jnp.dot(
        h, wr2_ref[...], preferred_element_type=_f32)


def _stage3(a1, r1, cntp, Wl2, Wr2, bl1, g1, be1):
  return pl.pallas_call(
      _stage3_body,
      grid=(NBLK,),
      in_specs=[
          pl.BlockSpec((NCHUNK, BN, CW), lambda i: (0, i, 0)),
          pl.BlockSpec((BN, T * H), lambda i: (i, 0)),
          pl.BlockSpec((BN, 16), lambda i: (i, 0)),
          pl.BlockSpec((H, H), lambda i: (0, 0)),
          pl.BlockSpec((H, H), lambda i: (0, 0)),
          pl.BlockSpec((1, H), lambda i: (0, 0)),
          pl.BlockSpec((1, H), lambda i: (0, 0)),
          pl.BlockSpec((1, H), lambda i: (0, 0)),
      ],
      out_specs=[
          pl.BlockSpec((BN, T * H), lambda i: (i, 0)),
          pl.BlockSpec((NCHUNK, BN, CW), lambda i: (0, i, 0)),
          pl.BlockSpec((BN, T * H), lambda i: (i, 0)),
      ],
      out_shape=[
          jax.ShapeDtypeStruct((N, T * H), _f32),
          jax.ShapeDtypeStruct((NCHUNK, NP, CW), _f32),
          jax.ShapeDtypeStruct((N, T * H), _f32),
      ],
  )(a1, r1, cntp, Wl2, Wr2, bl1, g1, be1)


def _stage5_body(a_ref, r2_ref, h_ref, cntp_ref, bl2_ref, g2_ref, be2_ref,
                 wi_ref, wh_ref, bi_ref, bh_ref, wc1_ref, bc1_ref, wc2_ref,
                 bc2_ref, out_ref):
  inv = _inv_cnt(cntp_ref[...])
  hts = []
  for t in range(T):
    pre = (a_ref[t] * inv + bl2_ref[...]
           + r2_ref[:, t * H:(t + 1) * H])
    h2 = _ln_relu(pre, g2_ref, be2_ref)
    hts.append(h2 + h_ref[:, t * H:(t + 1) * H])
  state = jnp.zeros((BN, H), _f32)
  for t in range(T):
    gi = jnp.dot(hts[t], wi_ref[...], preferred_element_type=_f32) + bi_ref[...]
    gh = jnp.dot(state, wh_ref[...], preferred_element_type=_f32) + bh_ref[...]
    r = jax.nn.sigmoid(gi[:, :H] + gh[:, :H])
    z = jax.nn.sigmoid(gi[:, H:2 * H] + gh[:, H:2 * H])
    n = jnp.tanh(gi[:, 2 * H:] + r * gh[:, 2 * H:])
    state = (1.0 - z) * n + z * state
  hc = jnp.maximum(
      jnp.dot(state, wc1_ref[...], preferred_element_type=_f32)
      + bc1_ref[...], 0.0)
  out_ref[...] = (jnp.dot(hc, wc2_ref[...], preferred_element_type=_f32)
                  + bc2_ref[...])


def _stage5(a2, r2, h, cntp, bl2, g2, be2, Wi, Wh, bi, bh, Wc1, bc1, Wc2p,
            bc2p):
  return pl.pallas_call(
      _stage5_body,
      grid=(NBLK,),
      in_specs=[
          pl.BlockSpec((NCHUNK, BN, CW), lambda i: (0, i, 0)),
          pl.BlockSpec((BN, T * H), lambda i: (i, 0)),
          pl.BlockSpec((BN, T * H), lambda i: (i, 0)),
          pl.BlockSpec((BN, 16), lambda i: (i, 0)),
          pl.BlockSpec((1, H), lambda i: (0, 0)),
          pl.BlockSpec((1, H), lambda i: (0, 0)),
          pl.BlockSpec((1, H), lambda i: (0, 0)),
          pl.BlockSpec((H, 3 * H), lambda i: (0, 0)),
          pl.BlockSpec((H, 3 * H), lambda i: (0, 0)),
          pl.BlockSpec((1, 3 * H), lambda i: (0, 0)),
          pl.BlockSpec((1, 3 * H), lambda i: (0, 0)),
          pl.BlockSpec((H, H // 2), lambda i: (0, 0)),
          pl.BlockSpec((1, H // 2), lambda i: (0, 0)),
          pl.BlockSpec((H // 2, 128), lambda i: (0, 0)),
          pl.BlockSpec((1, 128), lambda i: (0, 0)),
      ],
      out_specs=pl.BlockSpec((BN, 128), lambda i: (i, 0)),
      out_shape=jax.ShapeDtypeStruct((N, 128), _f32),
  )(a2, r2, h, cntp, bl2, g2, be2, Wi, Wh, bi, bh, Wc1, bc1, Wc2p, bc2p)


def kernel(x_seq, edge_index, Wl1, bl1, Wr1, g1, be1, Wl2, bl2, Wr2, g2, be2,
           Wi, Wh, bi, bh, Wc1, bc1, Wc2, bc2):
  src = edge_index[0]
  dst = edge_index[1]
  pad_src = jnp.concatenate([src, jnp.zeros((EP - E,), jnp.int32)])
  pad_dst = jnp.concatenate([dst, jnp.full((EP - E,), N, jnp.int32)])
  src4 = (pad_src[None, :]
          + (jnp.arange(NCHUNK, dtype=jnp.int32) * NP)[:, None]
          ).reshape(NCHUNK * E_ROWS, 128)
  dst2 = pad_dst.reshape(E_ROWS, 128)
  z2 = jnp.zeros((NP, CW), _f32)
  zc = jnp.zeros((NP, 16), _f32)
  o1 = jnp.ones((128, 16), _f32)

  y1, r1 = _stage1(x_seq, Wl1, Wr1)
  a1, cntp = _sc_agg_cnt(y1.reshape(NCHUNK * NP, CW), src4, dst2, z2, zc, o1)
  cntp = cntp[:NP]
  h, y2, r2 = _stage3(a1.reshape(NCHUNK, NP, CW), r1, cntp, Wl2, Wr2,
                      bl1[None, :], g1[None, :], be1[None, :])
  a2 = _sc_agg(y2.reshape(NCHUNK * NP, CW), src4, dst2, z2, zc, o1)
  Wc2p = jnp.zeros((H // 2, 128), _f32).at[:, :2].set(Wc2)
  bc2p = jnp.zeros((1, 128), _f32).at[:, :2].set(bc2[None, :])
  out = _stage5(a2.reshape(NCHUNK, NP, CW), r2, h, cntp, bl2[None, :],
                g2[None, :], be2[None, :], Wi, Wh, bi[None, :], bh[None, :],
                Wc1, bc1[None, :], Wc2p, bc2p)
  return out[:, :2]


# speedup vs baseline: 1.6579x; 1.6579x over previous
import functools

import jax
import jax.numpy as jnp
from jax import lax
from jax.experimental import pallas as pl
from jax.experimental.pallas import tpu as pltpu
from jax.experimental.pallas import tpu_sc as plsc

N = 10000
E = 320000
T = 8
F_IN = 128
H = 64

NP = 10112
EP = 327680
NCHUNK = 16
CW = 32
CPS = NCHUNK // 2
E_ROWS = EP // 128
NSUB = 16
RPT = NP // NSUB
ER_TILE = E_ROWS // NSUB
GROUPS = ER_TILE
GB = 128
NBUF = 4

BN = 1000
NBLK = N // BN

_f32 = jnp.float32


def _make_sc_agg(with_cnt):
  mesh = plsc.VectorSubcoreMesh(core_axis_name="c", subcore_axis_name="s")
  out_type = [jax.ShapeDtypeStruct((NCHUNK * NP, CW), _f32)]
  scratch = [
      pltpu.VMEM((ER_TILE, 128), jnp.int32),
      pltpu.VMEM((ER_TILE, 128), jnp.int32),
  ] + [pltpu.VMEM((GB, CW), _f32) for _ in range(NBUF)] + [
      pltpu.VMEM_SHARED((NP, CW), _f32),
      pltpu.VMEM_SHARED((NP, CW), _f32),
  ] + [pltpu.SemaphoreType.DMA for _ in range(NBUF)]
  if with_cnt:
    out_type.append(jax.ShapeDtypeStruct((2 * NP, 16), _f32))
    scratch += [
        pltpu.VMEM((128, 16), _f32),
        pltpu.VMEM_SHARED((NP, 16), _f32),
    ]

  def body(y_h, s2_h, d2_h, z2_h, zc_h, o1_h, *rest):
    if with_cnt:
      out_h, cnt_h = rest[0], rest[1]
      (src_sl, dst_sl), rows = rest[2:4], rest[4:4 + NBUF]
      tbl_sp, acc_sp = rest[4 + NBUF], rest[5 + NBUF]
      gsem = rest[6 + NBUF:6 + 2 * NBUF]
      o1_v, cnt_sp = rest[6 + 2 * NBUF], rest[7 + 2 * NBUF]
    else:
      out_h = rest[0]
      (src_sl, dst_sl), rows = rest[1:3], rest[3:3 + NBUF]
      tbl_sp, acc_sp = rest[3 + NBUF], rest[4 + NBUF]
      gsem = rest[5 + NBUF:5 + 2 * NBUF]
    cid = lax.axis_index("c")
    sid = lax.axis_index("s")
    row0 = sid * RPT
    pltpu.sync_copy(s2_h.at[pl.ds(sid * ER_TILE, ER_TILE)], src_sl)
    pltpu.sync_copy(d2_h.at[pl.ds(sid * ER_TILE, ER_TILE)], dst_sl)
    if with_cnt:
      pltpu.sync_copy(o1_h, o1_v)
      pltpu.sync_copy(zc_h.at[pl.ds(row0, RPT)], cnt_sp.at[pl.ds(row0, RPT)])

    def fire_gather(g, b):
      pltpu.async_copy(tbl_sp.at[src_sl.at[g, :]], rows[b], gsem[b])

    for q in range(CPS):
      chunk = cid * CPS + q
      pltpu.sync_copy(y_h.at[pl.ds(chunk * NP + row0, RPT)],
                      tbl_sp.at[pl.ds(row0, RPT)])
      pltpu.sync_copy(z2_h.at[pl.ds(row0, RPT)], acc_sp.at[pl.ds(row0, RPT)])
      plsc.subcore_barrier()

      for b in range(NBUF):
        fire_gather(b, b)

      def ring_body(gg, carry):
        for b in range(NBUF):
          g = gg * NBUF + b
          pltpu.make_async_copy(tbl_sp.at[pl.ds(0, GB)], rows[b],
                                gsem[b]).wait()
          pltpu.sync_copy(rows[b], acc_sp.at[dst_sl.at[g, :]], add=True)
          if with_cnt and q == 0:
            pltpu.sync_copy(o1_v, cnt_sp.at[dst_sl.at[g, :]], add=True)

          @pl.when(g + NBUF < GROUPS)
          def _():
            fire_gather(g + NBUF, b)
        return carry

      lax.fori_loop(0, GROUPS // NBUF, ring_body, 0)
      plsc.subcore_barrier()
      pltpu.sync_copy(acc_sp.at[pl.ds(row0, RPT)],
                      out_h.at[pl.ds(chunk * NP + row0, RPT)])
      if with_cnt and q == 0:
        pltpu.sync_copy(cnt_sp.at[pl.ds(row0, RPT)],
                        cnt_h.at[pl.ds(cid * NP + row0, RPT)])

  return functools.partial(
      pl.kernel, mesh=mesh, out_type=out_type, scratch_types=scratch,
      compiler_params=pltpu.CompilerParams(use_tc_tiling_on_sc=False))(body)


@functools.lru_cache(maxsize=None)
def _sc_agg_fn(with_cnt):
  return _make_sc_agg(with_cnt)


def _sc_agg_cnt(*args):
  return _sc_agg_fn(True)(*args)


def _sc_agg(*args):
  out = _sc_agg_fn(False)(*args)
  return out[0] if isinstance(out, (list, tuple)) else out


def _stage1_body(x_ref, wl_ref, wr_ref, y_ref, r_ref):
  for t in range(T):
    xt = x_ref[t]
    yt = jnp.dot(xt, wl_ref[...], preferred_element_type=_f32)
    y_ref[2 * t] = yt[:, :CW]
    y_ref[2 * t + 1] = yt[:, CW:]
    r_ref[:, t * H:(t + 1) * H] = jnp.dot(
        xt, wr_ref[...], preferred_element_type=_f32)


def _stage1(x_seq, Wl1, Wr1):
  return pl.pallas_call(
      _stage1_body,
      grid=(NBLK,),
      in_specs=[
          pl.BlockSpec((T, BN, F_IN), lambda i: (0, i, 0)),
          pl.BlockSpec((F_IN, H), lambda i: (0, 0)),
          pl.BlockSpec((F_IN, H), lambda i: (0, 0)),
      ],
      out_specs=[
          pl.BlockSpec((NCHUNK, BN, CW), lambda i: (0, i, 0)),
          pl.BlockSpec((BN, T * H), lambda i: (i, 0)),
      ],
      out_shape=[
          jax.ShapeDtypeStruct((NCHUNK, NP, CW), _f32),
          jax.ShapeDtypeStruct((N, T * H), _f32),
      ],
  )(x_seq, Wl1, Wr1)


def _inv_cnt(cntp):
  cnt16 = jnp.sum(cntp, axis=1)
  return (16.0 / jnp.maximum(cnt16, 16.0))[:, None]


def _ln_relu(pre, g_ref, b_ref):
  mu = jnp.mean(pre, axis=1, keepdims=True)
  var = jnp.mean((pre - mu) * (pre - mu), axis=1, keepdims=True)
  hn = (pre - mu) * lax.rsqrt(var + 1e-5) * g_ref[...] + b_ref[...]
  return jnp.maximum(hn, 0.0)


def _stage3_body(a_ref, r_ref, cntp_ref, wl2_ref, wr2_ref, bl1_ref, g1_ref,
                 be1_ref, h_ref, y2_ref, r2_ref):
  inv = _inv_cnt(cntp_ref[...])
  for t in range(T):
    aggt = jnp.concatenate([a_ref[2 * t], a_ref[2 * t + 1]], axis=1)
    pre = (aggt * inv + bl1_ref[...]
           + r_ref[:, t * H:(t + 1) * H])
    h = _ln_relu(pre, g1_ref, be1_ref)
    h_ref[:, t * H:(t + 1) * H] = h
    y2t = jnp.dot(h, wl2_ref[...], preferred_element_type=_f32)
    y2_ref[2 * t] = y2t[:, :CW]
    y2_ref[2 * t + 1] = y2t[:, CW:]
    r2_ref[:, t * H:(t + 1) * H] = jnp.dot(
        h, wr2_ref[...], preferred_element_type=_f32)


def _stage3(a1, r1, cntp, Wl2, Wr2, bl1, g1, be1):
  return pl.pallas_call(
      _stage3_body,
      grid=(NBLK,),
      in_specs=[
          pl.BlockSpec((NCHUNK, BN, CW), lambda i: (0, i, 0)),
          pl.BlockSpec((BN, T * H), lambda i: (i, 0)),
          pl.BlockSpec((BN, 16), lambda i: (i, 0)),
          pl.BlockSpec((H, H), lambda i: (0, 0)),
          pl.BlockSpec((H, H), lambda i: (0, 0)),
          pl.BlockSpec((1, H), lambda i: (0, 0)),
          pl.BlockSpec((1, H), lambda i: (0, 0)),
          pl.BlockSpec((1, H), lambda i: (0, 0)),
      ],
      out_specs=[
          pl.BlockSpec((BN, T * H), lambda i: (i, 0)),
          pl.BlockSpec((NCHUNK, BN, CW), lambda i: (0, i, 0)),
          pl.BlockSpec((BN, T * H), lambda i: (i, 0)),
      ],
      out_shape=[
          jax.ShapeDtypeStruct((N, T * H), _f32),
          jax.ShapeDtypeStruct((NCHUNK, NP, CW), _f32),
          jax.ShapeDtypeStruct((N, T * H), _f32),
      ],
  )(a1, r1, cntp, Wl2, Wr2, bl1, g1, be1)


def _stage5_body(a_ref, r2_ref, h_ref, cntp_ref, bl2_ref, g2_ref, be2_ref,
                 wi_ref, wh_ref, bi_ref, bh_ref, wc1_ref, bc1_ref, wc2_ref,
                 bc2_ref, out_ref):
  inv = _inv_cnt(cntp_ref[...])
  hts = []
  for t in range(T):
    aggt = jnp.concatenate([a_ref[2 * t], a_ref[2 * t + 1]], axis=1)
    pre = (aggt * inv + bl2_ref[...]
           + r2_ref[:, t * H:(t + 1) * H])
    h2 = _ln_relu(pre, g2_ref, be2_ref)
    hts.append(h2 + h_ref[:, t * H:(t + 1) * H])
  state = jnp.zeros((BN, H), _f32)
  for t in range(T):
    gi = jnp.dot(hts[t], wi_ref[...], preferred_element_type=_f32) + bi_ref[...]
    gh = jnp.dot(state, wh_ref[...], preferred_element_type=_f32) + bh_ref[...]
    r = jax.nn.sigmoid(gi[:, :H] + gh[:, :H])
    z = jax.nn.sigmoid(gi[:, H:2 * H] + gh[:, H:2 * H])
    n = jnp.tanh(gi[:, 2 * H:] + r * gh[:, 2 * H:])
    state = (1.0 - z) * n + z * state
  hc = jnp.maximum(
      jnp.dot(state, wc1_ref[...], preferred_element_type=_f32)
      + bc1_ref[...], 0.0)
  out_ref[...] = (jnp.dot(hc, wc2_ref[...], preferred_element_type=_f32)
                  + bc2_ref[...])


def _stage5(a2, r2, h, cntp, bl2, g2, be2, Wi, Wh, bi, bh, Wc1, bc1, Wc2p,
            bc2p):
  return pl.pallas_call(
      _stage5_body,
      grid=(NBLK,),
      in_specs=[
          pl.BlockSpec((NCHUNK, BN, CW), lambda i: (0, i, 0)),
          pl.BlockSpec((BN, T * H), lambda i: (i, 0)),
          pl.BlockSpec((BN, T * H), lambda i: (i, 0)),
          pl.BlockSpec((BN, 16), lambda i: (i, 0)),
          pl.BlockSpec((1, H), lambda i: (0, 0)),
          pl.BlockSpec((1, H), lambda i: (0, 0)),
          pl.BlockSpec((1, H), lambda i: (0, 0)),
          pl.BlockSpec((H, 3 * H), lambda i: (0, 0)),
          pl.BlockSpec((H, 3 * H), lambda i: (0, 0)),
          pl.BlockSpec((1, 3 * H), lambda i: (0, 0)),
          pl.BlockSpec((1, 3 * H), lambda i: (0, 0)),
          pl.BlockSpec((H, H // 2), lambda i: (0, 0)),
          pl.BlockSpec((1, H // 2), lambda i: (0, 0)),
          pl.BlockSpec((H // 2, 128), lambda i: (0, 0)),
          pl.BlockSpec((1, 128), lambda i: (0, 0)),
      ],
      out_specs=pl.BlockSpec((BN, 128), lambda i: (i, 0)),
      out_shape=jax.ShapeDtypeStruct((N, 128), _f32),
  )(a2, r2, h, cntp, bl2, g2, be2, Wi, Wh, bi, bh, Wc1, bc1, Wc2p, bc2p)


def kernel(x_seq, edge_index, Wl1, bl1, Wr1, g1, be1, Wl2, bl2, Wr2, g2, be2,
           Wi, Wh, bi, bh, Wc1, bc1, Wc2, bc2):
  src = edge_index[0]
  dst = edge_index[1]
  pad_src = jnp.concatenate([src, jnp.zeros((EP - E,), jnp.int32)])
  pad_dst = jnp.concatenate([dst, jnp.full((EP - E,), N, jnp.int32)])
  src2 = pad_src.reshape(E_ROWS, 128)
  dst2 = pad_dst.reshape(E_ROWS, 128)
  z2 = jnp.zeros((NP, CW), _f32)
  zc = jnp.zeros((NP, 16), _f32)
  o1 = jnp.ones((128, 16), _f32)

  y1, r1 = _stage1(x_seq, Wl1, Wr1)
  a1, cntp = _sc_agg_cnt(y1.reshape(NCHUNK * NP, CW), src2, dst2, z2, zc, o1)
  cntp = cntp[:NP]
  h, y2, r2 = _stage3(a1.reshape(NCHUNK, NP, CW), r1, cntp, Wl2, Wr2,
                      bl1[None, :], g1[None, :], be1[None, :])
  a2 = _sc_agg(y2.reshape(NCHUNK * NP, CW), src2, dst2, z2, zc, o1)
  Wc2p = jnp.zeros((H // 2, 128), _f32).at[:, :2].set(Wc2)
  bc2p = jnp.zeros((1, 128), _f32).at[:, :2].set(bc2[None, :])
  out = _stage5(a2.reshape(NCHUNK, NP, CW), r2, h, cntp, bl2[None, :],
                g2[None, :], be2[None, :], Wi, Wh, bi[None, :], bh[None, :],
                Wc1, bc1[None, :], Wc2p, bc2p)
  return out[:, :2]
